# baseline (speedup 1.0000x reference)
import functools
import jax
import jax.numpy as jnp
from jax import lax
from jax.experimental import pallas as pl
from jax.experimental.pallas import tpu as pltpu
from jax.experimental.pallas import tpu_sc as plsc

N = 100000
E = 6400000
NC = 2
NS = 16
NW = NC * NS
L = 16
CHUNK = 8000
NCH = E // CHUNK
SUBIT = (NCH + NW - 1) // NW + 1
MACRO = (SUBIT + 2) // 3
UNROLL = 4
NPT = 6272
NPAD = NS * NPT
HIMASK = jnp.int32(-65536)

_mesh = plsc.VectorSubcoreMesh(
    core_axis_name="c", subcore_axis_name="s", num_cores=NC)


@functools.partial(
    pl.kernel,
    out_type=jax.ShapeDtypeStruct((NC, NPAD), jnp.float32),
    mesh=_mesh,
    scratch_types=[
        pltpu.VMEM((N // 2,), jnp.int32),
        pltpu.VMEM((CHUNK,), jnp.int32),
        pltpu.VMEM((CHUNK,), jnp.int32),
        pltpu.VMEM((CHUNK,), jnp.int32),
        pltpu.VMEM((CHUNK,), jnp.int32),
        pltpu.VMEM((CHUNK,), jnp.int32),
        pltpu.VMEM((CHUNK,), jnp.int32),
        pltpu.VMEM((CHUNK,), jnp.float32),
        pltpu.VMEM((CHUNK,), jnp.float32),
        pltpu.VMEM((CHUNK,), jnp.float32),
        pltpu.VMEM_SHARED((NPAD,), jnp.float32),
        pltpu.SemaphoreType.DMA((3,)),
        pltpu.SemaphoreType.DMA((3,)),
    ],
    compiler_params=pltpu.CompilerParams(needs_layout_passes=False),
)
def _sc_scatter(eif_hbm, w_hbm, vals_hbm, out_hbm, vals_v,
                src_a, src_b, src_c, dst_a, dst_b, dst_c,
                wm_a, wm_b, wm_c,
                acc_sh, sem_in, sem_sc):
    srcs = (src_a, src_b, src_c)
    dsts = (dst_a, dst_b, dst_c)
    wms = (wm_a, wm_b, wm_c)
    cid = lax.axis_index("c")
    sid = lax.axis_index("s")
    wid = sid * NC + cid

    def _fire_in(c, j):
        base = pl.multiple_of(c * CHUNK, 8)
        dbase = pl.multiple_of(E + c * CHUNK, 8)
        pltpu.async_copy(eif_hbm.at[pl.ds(base, CHUNK)], srcs[j],
                         sem_in.at[j])
        pltpu.async_copy(eif_hbm.at[pl.ds(dbase, CHUNK)], dsts[j],
                         sem_in.at[j])
        pltpu.async_copy(w_hbm.at[pl.ds(base, CHUNK)], wms[j],
                         sem_in.at[j])

    def _wait_in(c, j):
        base = pl.multiple_of(c * CHUNK, 8)
        dbase = pl.multiple_of(E + c * CHUNK, 8)
        pltpu.make_async_copy(eif_hbm.at[pl.ds(base, CHUNK)], srcs[j],
                              sem_in.at[j]).wait()
        pltpu.make_async_copy(eif_hbm.at[pl.ds(dbase, CHUNK)], dsts[j],
                              sem_in.at[j]).wait()
        pltpu.make_async_copy(w_hbm.at[pl.ds(base, CHUNK)], wms[j],
                              sem_in.at[j]).wait()

    def _wait_scatter(j):
        pltpu.make_async_copy(wms[j], acc_sh.at[dsts[j]],
                              sem_sc.at[j]).wait()

    _fire_in(wid, 0)
    _fire_in(NW + wid, 1)

    zeros = jnp.zeros((L,), jnp.float32)

    def _z(i, carry):
        wm_c[pl.ds(i * L, L)] = zeros
        return carry

    lax.fori_loop(0, NPT // L, _z, 0)
    pltpu.sync_copy(wm_c.at[pl.ds(0, NPT)],
                    acc_sh.at[pl.ds(sid * NPT, NPT)])

    pltpu.sync_copy(vals_hbm, vals_v)

    plsc.subcore_barrier()

    def _macro(i, carry):
        for j in range(3):
            i3 = i * 3 + j
            c = i3 * NW + wid

            @pl.when(c < NCH)
            def _():
                _wait_in(c, j)

                def _g(g, acc):
                    for u in range(UNROLL):
                        o = g * (L * UNROLL) + u * L
                        idx = srcs[j][pl.ds(o, L)]
                        word = plsc.load_gather(vals_v, [idx >> 1])
                        odd = (idx & 1) == 1
                        bits = jnp.where(odd, word & HIMASK, word << 16)
                        v = plsc.bitcast(bits, jnp.float32)
                        wms[j][pl.ds(o, L)] = v * wms[j][pl.ds(o, L)]
                    return acc

                lax.fori_loop(0, CHUNK // (L * UNROLL), _g, 0)
                pltpu.async_copy(wms[j], acc_sh.at[dsts[j]],
                                 sem_sc.at[j], add=True)

            jp = (j + 2) % 3
            cprev = c - NW

            @pl.when((i3 >= 1) & (cprev < NCH))
            def _():
                _wait_scatter(jp)

            cnext = c + 2 * NW

            @pl.when(cnext < NCH)
            def _():
                _fire_in(cnext, jp)

        return carry

    lax.fori_loop(0, MACRO, _macro, 0)

    plsc.subcore_barrier()
    pltpu.sync_copy(acc_sh.at[pl.ds(sid * NPT, NPT)],
                    out_hbm.at[cid, pl.ds(sid * NPT, NPT)])


def _combine_body(x_ref, o_ref):
    s = jax.nn.sigmoid(x_ref[0] + x_ref[1])
    o_ref[...] = s[:N]


_combine = pl.pallas_call(
    _combine_body,
    out_shape=jax.ShapeDtypeStruct((N,), jnp.float32),
)


@jax.jit
def kernel(values, edge_index, edge_weight):
    packed = lax.bitcast_convert_type(
        values.astype(jnp.bfloat16).reshape(N // 2, 2), jnp.int32)
    partials = _sc_scatter(edge_index.reshape(2 * E), edge_weight, packed)
    return _combine(partials)
